# baseline (speedup 1.0000x reference)
import numpy as np
import jax
import jax.numpy as jnp
from jax.experimental import pallas as pl

_NUM_CLASSES = 80
_STRIDE = 16
_SCALES = [1.0]
_ASPECTS = [0.5, 1.0, 2.0]
_FH, _FW = 80, 80
_ENC = 0.1


def _anchors_flat():
    scales = np.array(_SCALES, dtype=np.float32) * _STRIDE
    aspects = np.array(_ASPECTS, dtype=np.float32)
    sizes = scales[:, None] * np.array([1.0, 1.0], dtype=np.float32)[None, :]
    ratios = np.stack([np.sqrt(aspects), 1.0 / np.sqrt(aspects)], axis=-1)
    sizes = (ratios[None, ...] * sizes[:, None, :]).reshape(-1, 2)
    layout = np.concatenate([np.zeros_like(sizes), sizes], axis=-1)
    vx = (np.arange(_FW, dtype=np.float32) + 0.5) * _STRIDE
    vy = (np.arange(_FH, dtype=np.float32) + 0.5) * _STRIDE
    vyg, vxg = np.meshgrid(vy, vx, indexing="ij")
    offsets = np.stack([vxg, vyg], axis=-1)
    anchors = np.tile(layout[None, None, :, :], (_FH, _FW, 1, 1))
    anchors[:, :, :, :2] += offsets[:, :, None, :]
    return anchors.reshape(-1)


_R = 19200
_W = 80
_RB = 1920
_BB = 960


def _body(s_ref, b_ref, a_ref, so_ref, bo_ref):
    i = pl.program_id(0)
    so_ref[...] = jax.nn.sigmoid(jnp.transpose(s_ref[...]))

    @pl.when(i == 0)
    def _decode():
        d = jnp.transpose(b_ref[...])
        a = a_ref[...]
        lane = jax.lax.broadcasted_iota(jnp.int32, d.shape, 1)
        lo = (lane % 4) < 2
        m = jnp.where(lo, _ENC, 2.0 * _ENC)
        t = d * m + m
        a2 = jnp.roll(a, -2, axis=1)
        c = t * a2 + a
        s = jnp.exp(t) * a
        bo_ref[...] = jnp.where(
            lo, c - 0.5 * jnp.roll(s, -2, axis=1),
            jnp.roll(c, 2, axis=1) + 0.5 * s)


def kernel(score, bbox):
    s2 = score.reshape(_R, _W)
    b2 = bbox.reshape(_BB, _W)
    anch = jnp.asarray(_anchors_flat().reshape(_W, _BB))
    so, bo = pl.pallas_call(
        _body,
        grid=(_R // _RB,),
        in_specs=[
            pl.BlockSpec((_RB, _W), lambda i: (i, 0)),
            pl.BlockSpec((_BB, _W), lambda i: (0, 0)),
            pl.BlockSpec((_W, _BB), lambda i: (0, 0)),
        ],
        out_specs=[
            pl.BlockSpec((_W, _RB), lambda i: (0, i)),
            pl.BlockSpec((_W, _BB), lambda i: (0, 0)),
        ],
        out_shape=[
            jax.ShapeDtypeStruct((_W, _R), jnp.float32),
            jax.ShapeDtypeStruct((_W, _BB), jnp.float32),
        ],
    )(s2, b2, anch)
    return so.reshape(_R, _NUM_CLASSES), bo.reshape(_R, 4)


# speedup vs baseline: 1.5731x; 1.5731x over previous
import numpy as np
import jax
import jax.numpy as jnp
from jax.experimental import pallas as pl

_NUM_CLASSES = 80
_STRIDE = 16
_SCALES = [1.0]
_ASPECTS = [0.5, 1.0, 2.0]
_FH, _FW = 80, 80
_ENC = 0.1


def _anchors_flat():
    scales = np.array(_SCALES, dtype=np.float32) * _STRIDE
    aspects = np.array(_ASPECTS, dtype=np.float32)
    sizes = scales[:, None] * np.array([1.0, 1.0], dtype=np.float32)[None, :]
    ratios = np.stack([np.sqrt(aspects), 1.0 / np.sqrt(aspects)], axis=-1)
    sizes = (ratios[None, ...] * sizes[:, None, :]).reshape(-1, 2)
    layout = np.concatenate([np.zeros_like(sizes), sizes], axis=-1)
    vx = (np.arange(_FW, dtype=np.float32) + 0.5) * _STRIDE
    vy = (np.arange(_FH, dtype=np.float32) + 0.5) * _STRIDE
    vyg, vxg = np.meshgrid(vy, vx, indexing="ij")
    offsets = np.stack([vxg, vyg], axis=-1)
    anchors = np.tile(layout[None, None, :, :], (_FH, _FW, 1, 1))
    anchors[:, :, :, :2] += offsets[:, :, None, :]
    return anchors.reshape(-1)


_C = 3 * _NUM_CLASSES
_CB = 24
_BC = 12


def _body(s_ref, b_ref, a_ref, so_ref, bo_ref):
    j = pl.program_id(0)
    so_ref[...] = jax.nn.sigmoid(jnp.transpose(s_ref[0], (2, 0, 1)))

    @pl.when(j == 0)
    def _decode():
        d = jnp.transpose(b_ref[0], (2, 0, 1))
        a = a_ref[...]
        lane = jax.lax.broadcasted_iota(jnp.int32, d.shape, 2)
        lo = (lane % 4) < 2
        m = jnp.where(lo, _ENC, 2.0 * _ENC)
        t = d * m + m
        a2 = jnp.roll(a, -2, axis=2)
        c = t * a2 + a
        s = jnp.exp(t) * a
        bo_ref[...] = jnp.where(
            lo, c - 0.5 * jnp.roll(s, -2, axis=2),
            jnp.roll(c, 2, axis=2) + 0.5 * s)


def kernel(score, bbox):
    anch = jnp.asarray(_anchors_flat().reshape(_FW, _BC, _FH))
    so, bo = pl.pallas_call(
        _body,
        grid=(_C // _CB,),
        in_specs=[
            pl.BlockSpec((1, _CB, _FH, _FW), lambda j: (0, j, 0, 0)),
            pl.BlockSpec((1, _BC, _FH, _FW), lambda j: (0, 0, 0, 0)),
            pl.BlockSpec((_FW, _BC, _FH), lambda j: (0, 0, 0)),
        ],
        out_specs=[
            pl.BlockSpec((_FW, _CB, _FH), lambda j: (0, j, 0)),
            pl.BlockSpec((_FW, _BC, _FH), lambda j: (0, 0, 0)),
        ],
        out_shape=[
            jax.ShapeDtypeStruct((_FW, _C, _FH), jnp.float32),
            jax.ShapeDtypeStruct((_FW, _BC, _FH), jnp.float32),
        ],
    )(score, bbox, anch)
    return so.reshape(_FW * _C, _NUM_CLASSES), bo.reshape(_FW * _BC * 20, 4)
